# baseline (speedup 1.0000x reference)
import functools

import jax
import jax.numpy as jnp
from jax import lax
from jax.experimental import pallas as pl
from jax.experimental.pallas import tpu as pltpu
from jax.experimental.pallas import tpu_sc as plsc


def _make_sc_kernel(B, F, V, D):
    info = plsc.get_sparse_core_info()
    NC, NS, L = info.num_cores, info.num_subcores, info.num_lanes
    NW = NC * NS

    assert D % L == 0 and B % NW == 0
    b_per_w = B // NW
    G = 64
    assert b_per_w % G == 0
    NCHUNK = b_per_w // G
    IPC = G * F
    assert IPC % 128 == 0
    NIDX = IPC // 128

    mesh = plsc.VectorSubcoreMesh(core_axis_name="c", subcore_axis_name="s")

    @functools.partial(
        pl.kernel,
        mesh=mesh,
        compiler_params=pltpu.CompilerParams(use_tc_tiling_on_sc=False),
        out_type=jax.ShapeDtypeStruct((B, D), jnp.float32),
        scratch_types=[
            pltpu.VMEM((IPC,), jnp.int32),
            pltpu.VMEM((NIDX, 128), jnp.int32),
            pltpu.VMEM((IPC, D), jnp.float32),
            pltpu.VMEM((G, D), jnp.float32),
            pltpu.SemaphoreType.DMA,
        ],
    )
    def k(x1_hbm, tbl_hbm, out_hbm, raw_v, gidx_v, rows_v, out_v, sem):
        wid = lax.axis_index("s") * NC + lax.axis_index("c")

        def chunk_body(g, carry):
            p_base = (wid * NCHUNK + g) * IPC
            s_base = (wid * NCHUNK + g) * G

            pltpu.sync_copy(x1_hbm.at[pl.ds(p_base, IPC)], raw_v)

            for j in range(NIDX):
                for kk in range(128 // L):
                    base = (j * 128 + kk * L) % F
                    f = lax.rem(lax.iota(jnp.int32, L) + base, F)
                    gidx_v[j, pl.ds(kk * L, L)] = (
                        raw_v[pl.ds(j * 128 + kk * L, L)] + f * V
                    )

            copies = [
                pltpu.async_copy(
                    tbl_hbm.at[gidx_v.at[j]],
                    rows_v.at[pl.ds(j * 128, 128)],
                    sem,
                )
                for j in range(NIDX)
            ]
            for c in copies:
                c.wait()

            def sample_body(s, c2):
                r0 = s * F
                lo = rows_v[r0, pl.ds(0, L)]
                hi = rows_v[r0, pl.ds(L, L)]
                for f in range(1, F):
                    lo = lo + rows_v[r0 + f, pl.ds(0, L)]
                    hi = hi + rows_v[r0 + f, pl.ds(L, L)]
                out_v[s, pl.ds(0, L)] = lo
                out_v[s, pl.ds(L, L)] = hi
                return c2

            lax.fori_loop(0, G, sample_body, 0)

            pltpu.sync_copy(out_v, out_hbm.at[pl.ds(s_base, G)])
            return carry

        lax.fori_loop(0, NCHUNK, chunk_body, 0)

    return k


def kernel(x, tables):
    B, F = x.shape
    Ft, V, D = tables.shape
    assert Ft == F
    x1 = x.astype(jnp.int32).reshape(B * F)
    flat_tables = tables.reshape(F * V, D)
    return _make_sc_kernel(B, F, V, D)(x1, flat_tables)
